# Initial kernel scaffold; baseline (speedup 1.0000x reference)
import jax
import jax.numpy as jnp
from jax.experimental import pallas as pl


def kernel(x, bias, fc, emb, W1, b1, W2, b2, W3, b3, W4, b4):
    raise NotImplementedError("write your pallas kernel here")


# speedup vs baseline: 14.7046x; 14.7046x over previous
import functools

import jax
import jax.numpy as jnp
from jax import lax
from jax.experimental import pallas as pl
from jax.experimental.pallas import tpu as pltpu
from jax.experimental.pallas import tpu_sc as plsc

B = 4096
F = 26
V = 1000
TOTAL = F * V
D = 16
IN_MLP = F * D
H = 400

NC, NS, L = 2, 16, 16
NW = NC * NS
RPW = B // NW
CPW = RPW * F
CHUNKS = CPW // RPW


def _sc_gather(x3, emb, fcf):
    mesh = plsc.VectorSubcoreMesh(core_axis_name="c", subcore_axis_name="s")

    @functools.partial(
        pl.kernel,
        out_type=[
            jax.ShapeDtypeStruct((B * F, D), jnp.float32),
            jax.ShapeDtypeStruct((NW, F, RPW), jnp.float32),
        ],
        mesh=mesh,
        compiler_params=pltpu.CompilerParams(
            needs_layout_passes=False, use_tc_tiling_on_sc=False),
        scratch_types=[
            pltpu.VMEM((F, RPW), jnp.int32),
            pltpu.VMEM((CPW, D), jnp.float32),
            pltpu.VMEM((TOTAL,), jnp.float32),
            pltpu.VMEM((F, RPW), jnp.float32),
            pltpu.SemaphoreType.DMA,
        ],
    )
    def k(x3_hbm, emb_hbm, fcf_hbm, rows_out, fcv_out, idx_v, rows_v, fc_v,
          fcv_v, sem):
        wid = lax.axis_index("s") * NC + lax.axis_index("c")
        pltpu.sync_copy(x3_hbm.at[wid], idx_v)
        cps = [
            pltpu.async_copy(
                emb_hbm.at[idx_v.at[j]],
                rows_v.at[pl.ds(j * RPW, RPW)],
                sem,
            )
            for j in range(F)
        ]
        pltpu.sync_copy(fcf_hbm, fc_v)
        iota = lax.iota(jnp.int32, L)

        def body(a, _):
            for kk in range(RPW // L):
                xv = idx_v[a, pl.ds(kk * L, L)]
                p = a * RPW + kk * L + iota
                field = p % F
                val = plsc.load_gather(fc_v, [xv + field * V])
                fcv_v[a, pl.ds(kk * L, L)] = val
            return _

        lax.fori_loop(0, F, body, None)
        for cp in cps:
            cp.wait()
        pltpu.sync_copy(rows_v, rows_out.at[pl.ds(wid * CPW, CPW)])
        pltpu.sync_copy(fcv_v, fcv_out.at[wid])

    return k(x3, emb, fcf)


def _tc_body(embed_ref, fcv_ref, w1, b1, w2, b2, w3, b3, w4, b4, bias,
             out_ref, s_acc):
    phase = pl.program_id(0)

    @pl.when(phase == 0)
    def _():
        @pl.when(pl.program_id(1) == 0)
        def _():
            s_acc[0] = 0.0

        e = embed_ref[...]
        rs = jnp.sum(e, axis=1)
        s_acc[0] += jnp.sum(rs * rs) - jnp.sum(e * e)

    @pl.when(phase == 1)
    def _():
        e = embed_ref[...]
        h = jnp.maximum(jnp.dot(e, w1[...], preferred_element_type=jnp.float32)
                        + b1[...], 0.0)
        h = jnp.maximum(jnp.dot(h, w2[...], preferred_element_type=jnp.float32)
                        + b2[...], 0.0)
        h = jnp.maximum(jnp.dot(h, w3[...], preferred_element_type=jnp.float32)
                        + b3[...], 0.0)
        mlp = jnp.sum(h * w4[...], axis=1) + b4[0, 0]
        lin = jnp.sum(fcv_ref[...], axis=1)
        fm = bias[0, 0] + lin + 0.5 * s_acc[0]
        out_ref[...] = jax.nn.sigmoid(fm + mlp)[:, None]


def _tc_mlp(embed, fcv26, bias, w1, b1, w2, b2, w3, b3, w4, b4):
    bt = 256
    nt = B // bt
    full = lambda shape: pl.BlockSpec(shape, lambda p, t: (0, 0))
    return pl.pallas_call(
        _tc_body,
        grid=(2, nt),
        in_specs=[
            pl.BlockSpec((bt, IN_MLP), lambda p, t: (t, 0)),
            pl.BlockSpec((bt, F), lambda p, t: (t, 0)),
            full((IN_MLP, H)),
            full((1, H)),
            full((H, H)),
            full((1, H)),
            full((H, H)),
            full((1, H)),
            full((1, H)),
            full((1, 1)),
            full((1, 1)),
        ],
        out_specs=pl.BlockSpec((bt, 1), lambda p, t: (t, 0)),
        out_shape=jax.ShapeDtypeStruct((B, 1), jnp.float32),
        scratch_shapes=[pltpu.SMEM((1,), jnp.float32)],
    )(embed, fcv26, w1, b1, w2, b2, w3, b3, w4, b4, bias)


def kernel(x, bias, fc, emb, W1, b1, W2, b2, W3, b3, W4, b4):
    x3 = x.astype(jnp.int32).reshape(NW, F, RPW)
    fcf = fc.reshape(TOTAL)
    rows, fcv = _sc_gather(x3, emb, fcf)
    embed = rows.reshape(B, IN_MLP)
    fcv26 = fcv.reshape(B, F)
    return _tc_mlp(
        embed, fcv26, bias.reshape(1, 1),
        W1, b1.reshape(1, H), W2, b2.reshape(1, H), W3, b3.reshape(1, H),
        W4.reshape(1, H), b4.reshape(1, 1),
    )
